# baseline (speedup 1.0000x reference)
import dataclasses
import functools

import jax
import jax.numpy as jnp
from jax import lax
from jax.experimental import pallas as pl
from jax.experimental.pallas import tpu as pltpu
from jax.experimental.pallas import tpu_sc as plsc

N = 10000
D = 128
E = 320000

NC = 2
NS = 16
L = 16
NW = NC * NS
C = 64
NCHUNKS = E // C
SLOTS = NCHUNKS // NW
LEFTOVER = NCHUNKS - SLOTS * NW
NP = 10240
RPS = NP // NS
ZC = 64


def _pre_body(h_ref, d_ref, w_ref, b_ref, pq_ref, hs_ref):
    w = w_ref[0, :]
    h = h_ref[...]
    d = d_ref[...]
    pq_ref[0, :] = jnp.sum(h * w[:D][None, :], axis=1) + b_ref[0, 0]
    pq_ref[1, :] = jnp.sum(h * w[D:][None, :], axis=1)
    hs_ref[...] = h * d[:, None]


def _precompute(h, d, gate_w, gate_b):
    return pl.pallas_call(
        _pre_body,
        out_shape=[
            jax.ShapeDtypeStruct((2, N), jnp.float32),
            jax.ShapeDtypeStruct((N, D), jnp.float32),
        ],
    )(h, d, gate_w, gate_b.reshape(1, 1))


def _sc_body(src_hbm, dst_hbm, pb_hbm, q_hbm, hs_hbm, out_hbm,
             pb_v, q_v,
             isrc0, isrc1, isrc2, isrc3, idst0, idst1, idst2, idst3,
             rows0, rows1, rows2, e_v, zsh,
             i0, i1, i2, i3, g0, g1, g2, s0, s1, s2):
    cid = lax.axis_index("c")
    sid = lax.axis_index("s")
    wid = sid * NC + cid
    isrc, idst = (isrc0, isrc1, isrc2, isrc3), (idst0, idst1, idst2, idst3)
    rows = (rows0, rows1, rows2)
    isem, gsem, ssem = (i0, i1, i2, i3), (g0, g1, g2), (s0, s1, s2)

    pltpu.sync_copy(pb_hbm, pb_v)
    pltpu.sync_copy(q_hbm, q_v)

    @pl.loop(0, ZC)
    def _(i):
        for j in range(D // L):
            rows0[i, pl.ds(j * L, L)] = jnp.zeros((L,), jnp.float32)

    @pl.loop(0, RPS // ZC)
    def _(k):
        pltpu.sync_copy(rows0, zsh.at[pl.ds(sid * RPS + k * ZC, ZC)])

    plsc.subcore_barrier()

    def edge_off(c):
        return (wid + NW * c) * C

    def idx_start(c, m):
        off = edge_off(c)
        pltpu.make_async_copy(src_hbm.at[pl.ds(off, C)], isrc[m], isem[m]).start()
        pltpu.make_async_copy(dst_hbm.at[pl.ds(off, C)], idst[m], isem[m]).start()

    def idx_wait(c, m):
        off = edge_off(c)
        pltpu.make_async_copy(src_hbm.at[pl.ds(off, C)], isrc[m], isem[m]).wait()
        pltpu.make_async_copy(dst_hbm.at[pl.ds(off, C)], idst[m], isem[m]).wait()

    def gather_start(b, m):
        pltpu.make_async_copy(hs_hbm.at[isrc[m]], rows[b], gsem[b]).start()

    def gather_wait(b, m):
        pltpu.make_async_copy(hs_hbm.at[isrc[m]], rows[b], gsem[b]).wait()

    def compute(b, m):
        for g in range(C // L):
            s16 = isrc[m][pl.ds(g * L, L)]
            d16 = idst[m][pl.ds(g * L, L)]
            x = plsc.load_gather(pb_v, [d16]) + plsc.load_gather(q_v, [s16])
            t = jnp.exp(x + x)
            e_v[pl.ds(g * L, L)] = 1.0 - 2.0 / (t + 1.0)

        @plsc.parallel_loop(0, C, unroll=8)
        def _(i):
            ev = plsc.load_gather(e_v, [jnp.full((L,), i, jnp.int32)])
            for j in range(D // L):
                rows[b][i, pl.ds(j * L, L)] = rows[b][i, pl.ds(j * L, L)] * ev

    def scatter_start(b, m):
        pltpu.async_copy(rows[b], zsh.at[idst[m]], ssem[b], add=True)

    def scatter_wait(b, m):
        pltpu.make_async_copy(rows[b], zsh.at[idst[m]], ssem[b]).wait()

    def slot(k, kk):
        if kk >= 2:
            scatter_wait((kk - 2) % 3, (kk - 2) % 4)
        idx_start(k + 2, (kk + 2) % 4)
        idx_wait(k + 1, (kk + 1) % 4)
        gather_start((kk + 1) % 3, (kk + 1) % 4)
        gather_wait(kk % 3, kk % 4)
        compute(kk % 3, kk % 4)
        scatter_start(kk % 3, kk % 4)

    idx_start(0, 0)
    idx_start(1, 1)
    idx_wait(0, 0)
    gather_start(0, 0)

    slot(0, 0)
    slot(1, 1)
    slot(2, 2)

    @pl.loop(0, 12)
    def _(p):
        for j in range(12):
            slot(3 + 12 * p + j, 3 + j)

    for k in range(147, 156):
        scatter_wait((k - 2) % 3, (k - 2) % 4)
        if k + 2 <= 155:
            idx_start(k + 2, (k + 2) % 4)
        if k + 1 <= 155:
            idx_wait(k + 1, (k + 1) % 4)
            gather_start((k + 1) % 3, (k + 1) % 4)
        gather_wait(k % 3, k % 4)
        compute(k % 3, k % 4)
        scatter_start(k % 3, k % 4)

    @pl.when(wid < LEFTOVER)
    def _():
        idx_start(156, 0)
        idx_wait(156, 0)
        gather_start(0, 0)
        gather_wait(0, 0)
        compute(0, 0)
        scatter_start(0, 0)
        scatter_wait(0, 0)

    scatter_wait(1, 2)
    scatter_wait(2, 3)
    plsc.subcore_barrier()
    pltpu.sync_copy(zsh.at[pl.ds(sid * RPS, RPS)],
                    out_hbm.at[cid, pl.ds(sid * RPS, RPS)])


def _sc_call(src, dst, pb, q, hs):
    mesh = plsc.VectorSubcoreMesh(core_axis_name="c", subcore_axis_name="s")
    cp = pltpu.CompilerParams()
    if "needs_layout_passes" in pltpu.CompilerParams.__dataclass_fields__:
        cp = dataclasses.replace(cp, needs_layout_passes=False)
    f = functools.partial(
        pl.kernel,
        out_type=jax.ShapeDtypeStruct((NC, NP, D), jnp.float32),
        mesh=mesh,
        compiler_params=cp,
        scratch_types=(
            [pltpu.VMEM((N,), jnp.float32)] * 2
            + [pltpu.VMEM((C,), jnp.int32)] * 8
            + [pltpu.VMEM((C, D), jnp.float32)] * 3
            + [pltpu.VMEM((C,), jnp.float32)]
            + [pltpu.VMEM_SHARED((NP, D), jnp.float32)]
            + [pltpu.SemaphoreType.DMA] * 10
        ),
    )(_sc_body)
    return f(src, dst, pb, q, hs)


def _sum_body(a_ref, b_ref, d_ref, o_ref):
    o_ref[...] = (a_ref[...] + b_ref[...]) * d_ref[...]


def _sum_partials(z0, z1, d):
    blk = pl.BlockSpec((1000, D), lambda i: (i, 0))
    dblk = pl.BlockSpec((1000, 1), lambda i: (i, 0))
    return pl.pallas_call(
        _sum_body,
        grid=(N // 1000,),
        in_specs=[blk, blk, dblk],
        out_specs=blk,
        out_shape=jax.ShapeDtypeStruct((N, D), jnp.float32),
    )(z0, z1, d.reshape(N, 1))


def kernel(h, d, edge_index, gate_w, gate_b):
    src = edge_index[0]
    dst = edge_index[1]
    pq, hs = _precompute(h, d, gate_w, gate_b)
    zp = _sc_call(src, dst, pq[0], pq[1], hs)
    return _sum_partials(zp[0], zp[1], d)


# speedup vs baseline: 1.0737x; 1.0737x over previous
import dataclasses
import functools

import jax
import jax.numpy as jnp
from jax import lax
from jax.experimental import pallas as pl
from jax.experimental.pallas import tpu as pltpu
from jax.experimental.pallas import tpu_sc as plsc

N = 10000
D = 128
E = 320000

NC = 2
NS = 16
L = 16
NW = NC * NS
C = 64
NCHUNKS = E // C
SLOTS = NCHUNKS // NW
LEFTOVER = NCHUNKS - SLOTS * NW
NP = 10240
RPS = NP // NS
ZC = 64


def _pre_body(h_ref, d_ref, w_ref, b_ref, pq_ref, hs_ref):
    w = w_ref[0, :]
    h = h_ref[...]
    d = d_ref[...]
    pq_ref[0, :] = jnp.sum(h * w[:D][None, :], axis=1) + b_ref[0, 0]
    pq_ref[1, :] = jnp.sum(h * w[D:][None, :], axis=1)
    hs_ref[...] = h * d[:, None]


def _precompute(h, d, gate_w, gate_b):
    return pl.pallas_call(
        _pre_body,
        out_shape=[
            jax.ShapeDtypeStruct((2, N), jnp.float32),
            jax.ShapeDtypeStruct((N, D), jnp.float32),
        ],
    )(h, d, gate_w, gate_b.reshape(1, 1))


def _sc_body(src_hbm, dst_hbm, pb_hbm, q_hbm, hs_hbm, out_hbm,
             pb_v, q_v,
             isrc0, isrc1, isrc2, isrc3, idst0, idst1, idst2, idst3,
             rows0, rows1, rows2, e_v, zsh,
             i0, i1, i2, i3, g0, g1, g2, s0, s1, s2):
    cid = lax.axis_index("c")
    sid = lax.axis_index("s")
    wid = sid * NC + cid
    isrc, idst = (isrc0, isrc1, isrc2, isrc3), (idst0, idst1, idst2, idst3)
    rows = (rows0, rows1, rows2)
    isem, gsem, ssem = (i0, i1, i2, i3), (g0, g1, g2), (s0, s1, s2)

    pltpu.sync_copy(pb_hbm, pb_v)
    pltpu.sync_copy(q_hbm, q_v)

    @pl.loop(0, ZC)
    def _(i):
        for j in range(D // L):
            rows0[i, pl.ds(j * L, L)] = jnp.zeros((L,), jnp.float32)

    @pl.loop(0, RPS // ZC)
    def _(k):
        pltpu.sync_copy(rows0, zsh.at[pl.ds(sid * RPS + k * ZC, ZC)])

    plsc.subcore_barrier()

    def edge_off(c):
        return (wid + NW * c) * C

    def idx_start(c, m):
        off = edge_off(c)
        pltpu.make_async_copy(src_hbm.at[pl.ds(off, C)], isrc[m], isem[m]).start()
        pltpu.make_async_copy(dst_hbm.at[pl.ds(off, C)], idst[m], isem[m]).start()

    def idx_wait(c, m):
        off = edge_off(c)
        pltpu.make_async_copy(src_hbm.at[pl.ds(off, C)], isrc[m], isem[m]).wait()
        pltpu.make_async_copy(dst_hbm.at[pl.ds(off, C)], idst[m], isem[m]).wait()

    def gather_start(b, m):
        pltpu.make_async_copy(hs_hbm.at[isrc[m]], rows[b], gsem[b]).start()

    def gather_wait(b, m):
        pltpu.make_async_copy(hs_hbm.at[isrc[m]], rows[b], gsem[b]).wait()

    def compute(b, m):
        for g in range(C // L):
            s16 = isrc[m][pl.ds(g * L, L)]
            d16 = idst[m][pl.ds(g * L, L)]
            x = plsc.load_gather(pb_v, [d16]) + plsc.load_gather(q_v, [s16])
            t = jnp.exp(x + x)
            e_v[pl.ds(g * L, L)] = 1.0 - 2.0 / (t + 1.0)

        @plsc.parallel_loop(0, C, unroll=2)
        def _(i):
            ev = plsc.load_gather(e_v, [jnp.full((L,), i, jnp.int32)])
            for j in range(D // L):
                rows[b][i, pl.ds(j * L, L)] = rows[b][i, pl.ds(j * L, L)] * ev

    def scatter_start(b, m):
        pltpu.async_copy(rows[b], zsh.at[idst[m]], ssem[b], add=True)

    def scatter_wait(b, m):
        pltpu.make_async_copy(rows[b], zsh.at[idst[m]], ssem[b]).wait()

    def slot(k, kk):
        if kk >= 2:
            scatter_wait((kk - 2) % 3, (kk - 2) % 4)
        idx_start(k + 2, (kk + 2) % 4)
        idx_wait(k + 1, (kk + 1) % 4)
        gather_start((kk + 1) % 3, (kk + 1) % 4)
        gather_wait(kk % 3, kk % 4)
        compute(kk % 3, kk % 4)
        scatter_start(kk % 3, kk % 4)

    idx_start(0, 0)
    idx_start(1, 1)
    idx_wait(0, 0)
    gather_start(0, 0)

    slot(0, 0)
    slot(1, 1)
    slot(2, 2)

    @pl.loop(0, 12)
    def _(p):
        for j in range(12):
            slot(3 + 12 * p + j, 3 + j)

    for k in range(147, 156):
        scatter_wait((k - 2) % 3, (k - 2) % 4)
        if k + 2 <= 155:
            idx_start(k + 2, (k + 2) % 4)
        if k + 1 <= 155:
            idx_wait(k + 1, (k + 1) % 4)
            gather_start((k + 1) % 3, (k + 1) % 4)
        gather_wait(k % 3, k % 4)
        compute(k % 3, k % 4)
        scatter_start(k % 3, k % 4)

    @pl.when(wid < LEFTOVER)
    def _():
        idx_start(156, 0)
        idx_wait(156, 0)
        gather_start(0, 0)
        gather_wait(0, 0)
        compute(0, 0)
        scatter_start(0, 0)
        scatter_wait(0, 0)

    scatter_wait(1, 2)
    scatter_wait(2, 3)
    plsc.subcore_barrier()
    pltpu.sync_copy(zsh.at[pl.ds(sid * RPS, RPS)],
                    out_hbm.at[cid, pl.ds(sid * RPS, RPS)])


def _sc_call(src, dst, pb, q, hs):
    mesh = plsc.VectorSubcoreMesh(core_axis_name="c", subcore_axis_name="s")
    cp = pltpu.CompilerParams()
    if "needs_layout_passes" in pltpu.CompilerParams.__dataclass_fields__:
        cp = dataclasses.replace(cp, needs_layout_passes=False)
    f = functools.partial(
        pl.kernel,
        out_type=jax.ShapeDtypeStruct((NC, NP, D), jnp.float32),
        mesh=mesh,
        compiler_params=cp,
        scratch_types=(
            [pltpu.VMEM((N,), jnp.float32)] * 2
            + [pltpu.VMEM((C,), jnp.int32)] * 8
            + [pltpu.VMEM((C, D), jnp.float32)] * 3
            + [pltpu.VMEM((C,), jnp.float32)]
            + [pltpu.VMEM_SHARED((NP, D), jnp.float32)]
            + [pltpu.SemaphoreType.DMA] * 10
        ),
    )(_sc_body)
    return f(src, dst, pb, q, hs)


def _sum_body(a_ref, b_ref, d_ref, o_ref):
    o_ref[...] = (a_ref[...] + b_ref[...]) * d_ref[...]


def _sum_partials(z0, z1, d):
    blk = pl.BlockSpec((1000, D), lambda i: (i, 0))
    dblk = pl.BlockSpec((1000, 1), lambda i: (i, 0))
    return pl.pallas_call(
        _sum_body,
        grid=(N // 1000,),
        in_specs=[blk, blk, dblk],
        out_specs=blk,
        out_shape=jax.ShapeDtypeStruct((N, D), jnp.float32),
    )(z0, z1, d.reshape(N, 1))


def kernel(h, d, edge_index, gate_w, gate_b):
    src = edge_index[0]
    dst = edge_index[1]
    pq, hs = _precompute(h, d, gate_w, gate_b)
    zp = _sc_call(src, dst, pq[0], pq[1], hs)
    return _sum_partials(zp[0], zp[1], d)


# speedup vs baseline: 1.1245x; 1.0473x over previous
import dataclasses
import functools

import jax
import jax.numpy as jnp
from jax import lax
from jax.experimental import pallas as pl
from jax.experimental.pallas import tpu as pltpu
from jax.experimental.pallas import tpu_sc as plsc

N = 10000
D = 128
E = 320000

NC = 2
NS = 16
L = 16
NW = NC * NS
C = 64
NCHUNKS = E // C
SLOTS = NCHUNKS // NW
LEFTOVER = NCHUNKS - SLOTS * NW
NP = 10240
RPS = NP // NS
ZC = 64


def _pre_body(h_ref, d_ref, w_ref, b_ref, pq_ref, hs_ref):
    w = w_ref[0, :]
    h = h_ref[...]
    d = d_ref[...]
    pq_ref[0, :] = jnp.sum(h * w[:D][None, :], axis=1) + b_ref[0, 0]
    pq_ref[1, :] = jnp.sum(h * w[D:][None, :], axis=1)
    hs_ref[...] = h * d[:, None]


def _precompute(h, d, gate_w, gate_b):
    return pl.pallas_call(
        _pre_body,
        out_shape=[
            jax.ShapeDtypeStruct((2, N), jnp.float32),
            jax.ShapeDtypeStruct((N, D), jnp.float32),
        ],
    )(h, d, gate_w, gate_b.reshape(1, 1))


def _sc_body(ei_hbm, pb_hbm, q_hbm, hs_hbm, out_hbm,
             pb_v, q_v,
             isrc0, isrc1, isrc2, isrc3, idst0, idst1, idst2, idst3,
             rows0, rows1, rows2, e_v, zsh,
             i0, i1, i2, i3, g0, g1, g2, s0, s1, s2):
    cid = lax.axis_index("c")
    sid = lax.axis_index("s")
    wid = sid * NC + cid
    isrc, idst = (isrc0, isrc1, isrc2, isrc3), (idst0, idst1, idst2, idst3)
    rows = (rows0, rows1, rows2)
    isem, gsem, ssem = (i0, i1, i2, i3), (g0, g1, g2), (s0, s1, s2)

    pltpu.sync_copy(pb_hbm, pb_v)
    pltpu.sync_copy(q_hbm, q_v)

    @pl.loop(0, ZC)
    def _(i):
        for j in range(D // L):
            rows0[i, pl.ds(j * L, L)] = jnp.zeros((L,), jnp.float32)

    @pl.loop(0, RPS // ZC)
    def _(k):
        pltpu.sync_copy(rows0, zsh.at[pl.ds(sid * RPS + k * ZC, ZC)])

    plsc.subcore_barrier()

    def edge_off(c):
        return (wid + NW * c) * C

    def idx_start(c, m):
        off = edge_off(c)
        pltpu.make_async_copy(ei_hbm.at[pl.ds(off, C)], isrc[m], isem[m]).start()
        pltpu.make_async_copy(ei_hbm.at[pl.ds(E + off, C)], idst[m], isem[m]).start()

    def idx_wait(c, m):
        off = edge_off(c)
        pltpu.make_async_copy(ei_hbm.at[pl.ds(off, C)], isrc[m], isem[m]).wait()
        pltpu.make_async_copy(ei_hbm.at[pl.ds(E + off, C)], idst[m], isem[m]).wait()

    def gather_start(b, m):
        pltpu.make_async_copy(hs_hbm.at[isrc[m]], rows[b], gsem[b]).start()

    def gather_wait(b, m):
        pltpu.make_async_copy(hs_hbm.at[isrc[m]], rows[b], gsem[b]).wait()

    def compute(b, m):
        for g in range(C // L):
            s16 = isrc[m][pl.ds(g * L, L)]
            d16 = idst[m][pl.ds(g * L, L)]
            x = plsc.load_gather(pb_v, [d16]) + plsc.load_gather(q_v, [s16])
            t = jnp.exp(x + x)
            e_v[pl.ds(g * L, L)] = 1.0 - 2.0 / (t + 1.0)

        @plsc.parallel_loop(0, C, unroll=2)
        def _(i):
            ev = plsc.load_gather(e_v, [jnp.full((L,), i, jnp.int32)])
            for j in range(D // L):
                rows[b][i, pl.ds(j * L, L)] = rows[b][i, pl.ds(j * L, L)] * ev

    def scatter_start(b, m):
        pltpu.async_copy(rows[b], zsh.at[idst[m]], ssem[b], add=True)

    def scatter_wait(b, m):
        pltpu.make_async_copy(rows[b], zsh.at[idst[m]], ssem[b]).wait()

    def slot(k, kk):
        if kk >= 2:
            scatter_wait((kk - 2) % 3, (kk - 2) % 4)
        idx_start(k + 2, (kk + 2) % 4)
        idx_wait(k + 1, (kk + 1) % 4)
        gather_start((kk + 1) % 3, (kk + 1) % 4)
        gather_wait(kk % 3, kk % 4)
        compute(kk % 3, kk % 4)
        scatter_start(kk % 3, kk % 4)

    idx_start(0, 0)
    idx_start(1, 1)
    idx_wait(0, 0)
    gather_start(0, 0)

    slot(0, 0)
    slot(1, 1)
    slot(2, 2)

    @pl.loop(0, 12)
    def _(p):
        for j in range(12):
            slot(3 + 12 * p + j, 3 + j)

    for k in range(147, 156):
        scatter_wait((k - 2) % 3, (k - 2) % 4)
        if k + 2 <= 155:
            idx_start(k + 2, (k + 2) % 4)
        if k + 1 <= 155:
            idx_wait(k + 1, (k + 1) % 4)
            gather_start((k + 1) % 3, (k + 1) % 4)
        gather_wait(k % 3, k % 4)
        compute(k % 3, k % 4)
        scatter_start(k % 3, k % 4)

    @pl.when(wid < LEFTOVER)
    def _():
        idx_start(156, 0)
        idx_wait(156, 0)
        gather_start(0, 0)
        gather_wait(0, 0)
        compute(0, 0)
        scatter_start(0, 0)
        scatter_wait(0, 0)

    scatter_wait(1, 2)
    scatter_wait(2, 3)
    plsc.subcore_barrier()
    pltpu.sync_copy(zsh.at[pl.ds(sid * RPS, RPS)],
                    out_hbm.at[cid, pl.ds(sid * RPS, RPS)])


def _sc_call(ei, pb, q, hs):
    mesh = plsc.VectorSubcoreMesh(core_axis_name="c", subcore_axis_name="s")
    cp = pltpu.CompilerParams()
    if "needs_layout_passes" in pltpu.CompilerParams.__dataclass_fields__:
        cp = dataclasses.replace(cp, needs_layout_passes=False)
    f = functools.partial(
        pl.kernel,
        out_type=jax.ShapeDtypeStruct((NC, NP, D), jnp.float32),
        mesh=mesh,
        compiler_params=cp,
        scratch_types=(
            [pltpu.VMEM((N,), jnp.float32)] * 2
            + [pltpu.VMEM((C,), jnp.int32)] * 8
            + [pltpu.VMEM((C, D), jnp.float32)] * 3
            + [pltpu.VMEM((C,), jnp.float32)]
            + [pltpu.VMEM_SHARED((NP, D), jnp.float32)]
            + [pltpu.SemaphoreType.DMA] * 10
        ),
    )(_sc_body)
    return f(ei, pb, q, hs)


def _sum_body(a_ref, b_ref, d_ref, o_ref):
    o_ref[...] = (a_ref[...] + b_ref[...]) * d_ref[...]


def _sum_partials(z0, z1, d):
    blk = pl.BlockSpec((1000, D), lambda i: (i, 0))
    dblk = pl.BlockSpec((1000, 1), lambda i: (i, 0))
    return pl.pallas_call(
        _sum_body,
        grid=(N // 1000,),
        in_specs=[blk, blk, dblk],
        out_specs=blk,
        out_shape=jax.ShapeDtypeStruct((N, D), jnp.float32),
    )(z0, z1, d.reshape(N, 1))


def kernel(h, d, edge_index, gate_w, gate_b):
    pq, hs = _precompute(h, d, gate_w, gate_b)
    zp = _sc_call(edge_index.reshape(2 * E), pq[0], pq[1], hs)
    return _sum_partials(zp[0], zp[1], d)


# speedup vs baseline: 1.1655x; 1.0364x over previous
import dataclasses
import functools

import jax
import jax.numpy as jnp
from jax import lax
from jax.experimental import pallas as pl
from jax.experimental.pallas import tpu as pltpu
from jax.experimental.pallas import tpu_sc as plsc

N = 10000
D = 128
E = 320000

NC = 2
NS = 16
L = 16
NW = NC * NS
C = 64
NCHUNKS = E // C
SLOTS = NCHUNKS // NW
LEFTOVER = NCHUNKS - SLOTS * NW
NP = 10240
RPS = NP // NS
ZC = 64


def _pre_body(h_ref, d_ref, w_ref, b_ref, pq_ref, hs_ref):
    w = w_ref[0, :]
    h = h_ref[...]
    d = d_ref[...]
    pq_ref[0, :] = jnp.sum(h * w[:D][None, :], axis=1) + b_ref[0, 0]
    pq_ref[1, :] = jnp.sum(h * w[D:][None, :], axis=1)
    hs_ref[...] = h * d[:, None]


def _precompute(h, d, gate_w, gate_b):
    return pl.pallas_call(
        _pre_body,
        out_shape=[
            jax.ShapeDtypeStruct((2, N), jnp.float32),
            jax.ShapeDtypeStruct((N, D), jnp.float32),
        ],
    )(h, d, gate_w, gate_b.reshape(1, 1))


def _sc_body(ei_hbm, pq_hbm, hs_hbm, out_hbm,
             pb_v, q_v,
             isrc0, isrc1, isrc2, isrc3, idst0, idst1, idst2, idst3,
             rows0, rows1, rows2, e_v, zsh,
             i0, i1, i2, i3, g0, g1, g2, s0, s1, s2):
    cid = lax.axis_index("c")
    sid = lax.axis_index("s")
    wid = sid * NC + cid
    isrc, idst = (isrc0, isrc1, isrc2, isrc3), (idst0, idst1, idst2, idst3)
    rows = (rows0, rows1, rows2)
    isem, gsem, ssem = (i0, i1, i2, i3), (g0, g1, g2), (s0, s1, s2)

    pltpu.sync_copy(pq_hbm.at[pl.ds(0, N)], pb_v)
    pltpu.sync_copy(pq_hbm.at[pl.ds(N, N)], q_v)

    @pl.loop(0, ZC)
    def _(i):
        for j in range(D // L):
            rows0[i, pl.ds(j * L, L)] = jnp.zeros((L,), jnp.float32)

    @pl.loop(0, RPS // ZC)
    def _(k):
        pltpu.sync_copy(rows0, zsh.at[pl.ds(sid * RPS + k * ZC, ZC)])

    plsc.subcore_barrier()

    def edge_off(c):
        return (wid + NW * c) * C

    def idx_start(c, m):
        off = edge_off(c)
        pltpu.make_async_copy(ei_hbm.at[pl.ds(off, C)], isrc[m], isem[m]).start()
        pltpu.make_async_copy(ei_hbm.at[pl.ds(E + off, C)], idst[m], isem[m]).start()

    def idx_wait(c, m):
        off = edge_off(c)
        pltpu.make_async_copy(ei_hbm.at[pl.ds(off, C)], isrc[m], isem[m]).wait()
        pltpu.make_async_copy(ei_hbm.at[pl.ds(E + off, C)], idst[m], isem[m]).wait()

    def gather_start(b, m):
        pltpu.make_async_copy(hs_hbm.at[isrc[m]], rows[b], gsem[b]).start()

    def gather_wait(b, m):
        pltpu.make_async_copy(hs_hbm.at[isrc[m]], rows[b], gsem[b]).wait()

    def compute(b, m):
        for g in range(C // L):
            s16 = isrc[m][pl.ds(g * L, L)]
            d16 = idst[m][pl.ds(g * L, L)]
            x = plsc.load_gather(pb_v, [d16]) + plsc.load_gather(q_v, [s16])
            t = jnp.exp(x + x)
            e_v[pl.ds(g * L, L)] = 1.0 - 2.0 / (t + 1.0)

        @plsc.parallel_loop(0, C, unroll=2)
        def _(i):
            ev = plsc.load_gather(e_v, [jnp.full((L,), i, jnp.int32)])
            for j in range(D // L):
                rows[b][i, pl.ds(j * L, L)] = rows[b][i, pl.ds(j * L, L)] * ev

    def scatter_start(b, m):
        pltpu.async_copy(rows[b], zsh.at[idst[m]], ssem[b], add=True)

    def scatter_wait(b, m):
        pltpu.make_async_copy(rows[b], zsh.at[idst[m]], ssem[b]).wait()

    def slot(k, kk):
        if kk >= 2:
            scatter_wait((kk - 2) % 3, (kk - 2) % 4)
        idx_start(k + 2, (kk + 2) % 4)
        idx_wait(k + 1, (kk + 1) % 4)
        gather_start((kk + 1) % 3, (kk + 1) % 4)
        gather_wait(kk % 3, kk % 4)
        compute(kk % 3, kk % 4)
        scatter_start(kk % 3, kk % 4)

    idx_start(0, 0)
    idx_start(1, 1)
    idx_wait(0, 0)
    gather_start(0, 0)

    slot(0, 0)
    slot(1, 1)
    slot(2, 2)

    @pl.loop(0, 12)
    def _(p):
        for j in range(12):
            slot(3 + 12 * p + j, 3 + j)

    for k in range(147, 156):
        scatter_wait((k - 2) % 3, (k - 2) % 4)
        if k + 2 <= 155:
            idx_start(k + 2, (k + 2) % 4)
        if k + 1 <= 155:
            idx_wait(k + 1, (k + 1) % 4)
            gather_start((k + 1) % 3, (k + 1) % 4)
        gather_wait(k % 3, k % 4)
        compute(k % 3, k % 4)
        scatter_start(k % 3, k % 4)

    @pl.when(wid < LEFTOVER)
    def _():
        idx_start(156, 0)
        idx_wait(156, 0)
        gather_start(0, 0)
        gather_wait(0, 0)
        compute(0, 0)
        scatter_start(0, 0)
        scatter_wait(0, 0)

    scatter_wait(1, 2)
    scatter_wait(2, 3)
    plsc.subcore_barrier()
    pltpu.sync_copy(zsh.at[pl.ds(sid * RPS, RPS)],
                    out_hbm.at[cid, pl.ds(sid * RPS, RPS)])


def _sc_call(ei, pq, hs):
    mesh = plsc.VectorSubcoreMesh(core_axis_name="c", subcore_axis_name="s")
    cp = pltpu.CompilerParams()
    if "needs_layout_passes" in pltpu.CompilerParams.__dataclass_fields__:
        cp = dataclasses.replace(cp, needs_layout_passes=False)
    f = functools.partial(
        pl.kernel,
        out_type=jax.ShapeDtypeStruct((NC, NP, D), jnp.float32),
        mesh=mesh,
        compiler_params=cp,
        scratch_types=(
            [pltpu.VMEM((N,), jnp.float32)] * 2
            + [pltpu.VMEM((C,), jnp.int32)] * 8
            + [pltpu.VMEM((C, D), jnp.float32)] * 3
            + [pltpu.VMEM((C,), jnp.float32)]
            + [pltpu.VMEM_SHARED((NP, D), jnp.float32)]
            + [pltpu.SemaphoreType.DMA] * 10
        ),
    )(_sc_body)
    return f(ei, pq, hs)


def _sum_body(a_ref, b_ref, d_ref, o_ref):
    o_ref[...] = (a_ref[0] + b_ref[0]) * d_ref[...]


def _sum_partials(zp, d):
    blk0 = pl.BlockSpec((1, 1000, D), lambda i: (0, i, 0))
    blk1 = pl.BlockSpec((1, 1000, D), lambda i: (1, i, 0))
    dblk = pl.BlockSpec((1000, 1), lambda i: (i, 0))
    oblk = pl.BlockSpec((1000, D), lambda i: (i, 0))
    return pl.pallas_call(
        _sum_body,
        grid=(N // 1000,),
        in_specs=[blk0, blk1, dblk],
        out_specs=oblk,
        out_shape=jax.ShapeDtypeStruct((N, D), jnp.float32),
    )(zp, zp, d.reshape(N, 1))


def kernel(h, d, edge_index, gate_w, gate_b):
    pq, hs = _precompute(h, d, gate_w, gate_b)
    zp = _sc_call(edge_index.reshape(2 * E), pq.reshape(2 * N), hs)
    return _sum_partials(zp, d)
